# baseline (speedup 1.0000x reference)
import functools

import jax
import jax.numpy as jnp
from jax import lax
from jax.experimental import pallas as pl
from jax.experimental.pallas import tpu as pltpu
from jax.experimental.pallas import tpu_sc as plsc

NB = 8192
CD = 256
N = 16
T = 512
NT = N * T

CB_BLK = 1024
NCB = NB // CB_BLK

NTILES = 32
TOK_PER_TILE = NT // NTILES
NJ = TOK_PER_TILE // 128


GRP = 4096
NGRP = 2
BIG = 3.0e38


def _bf16(v):
    return v.astype(jnp.bfloat16).astype(jnp.float32)


def _dist_argmin_body(x_ref, cb_ref, idx_ref, minv, mini):
    cb_i = pl.program_id(1)
    x_blk = x_ref[0]
    cb_blk = cb_ref[...]
    m = jnp.dot(cb_blk.astype(jnp.bfloat16), x_blk.astype(jnp.bfloat16),
                preferred_element_type=jnp.float32)
    xn = jnp.sum(x_blk * x_blk, axis=0, keepdims=True)
    kn = jnp.sum(cb_blk * cb_blk, axis=1, keepdims=True)
    d = (xn - 2.0 * m) + kn
    rows = lax.broadcasted_iota(jnp.int32, d.shape, 0) + cb_i * CB_BLK

    @pl.when(cb_i == 0)
    def _():
        minv[...] = jnp.full((NGRP, T), BIG, jnp.float32)
        mini[...] = jnp.zeros((NGRP, T), jnp.int32)

    lo = 0
    for k in range(NCB):
        hi = lo + CB_BLK
        g_lo, g_hi = lo // GRP, min((hi - 1) // GRP, NGRP - 1)

        @pl.when(cb_i == k)
        def _(g_lo=g_lo, g_hi=g_hi, lo=lo, hi=hi):
            for g in range(g_lo, g_hi + 1):
                if g_lo == g_hi:
                    dg = d
                else:
                    in_g = ((rows >= g * GRP) &
                            (rows < min((g + 1) * GRP, NB)))
                    dg = jnp.where(in_g, d, BIG)
                bmin = jnp.min(dg, axis=0, keepdims=True)
                bidx = jnp.min(jnp.where(dg == bmin, rows, NB), axis=0,
                               keepdims=True)
                better = bmin < minv[g:g + 1, :]
                minv[g:g + 1, :] = jnp.where(better, bmin, minv[g:g + 1, :])
                mini[g:g + 1, :] = jnp.where(better, bidx, mini[g:g + 1, :])

        lo += CB_BLK

    @pl.when(cb_i == NCB - 1)
    def _():
        v = _bf16(minv[0:1, :])
        i = mini[0:1, :]
        for g in range(1, NGRP):
            take = minv[g:g + 1, :] < v
            v = _bf16(jnp.where(take, minv[g:g + 1, :], v))
            i = jnp.where(take, mini[g:g + 1, :], i)
        idx_ref[0] = i


def _stage_argmin(x, codebook):
    return pl.pallas_call(
        _dist_argmin_body,
        grid=(N, NCB),
        in_specs=[
            pl.BlockSpec((1, CD, T), lambda n, c: (n, 0, 0)),
            pl.BlockSpec((CB_BLK, CD), lambda n, c: (c, 0)),
        ],
        out_specs=pl.BlockSpec((1, 1, T), lambda n, c: (n, 0, 0)),
        out_shape=jax.ShapeDtypeStruct((N, 1, T), jnp.int32),
        scratch_shapes=[
            pltpu.VMEM((NGRP, T), jnp.float32),
            pltpu.VMEM((NGRP, T), jnp.int32),
        ],
    )(x, codebook)


def _sc_body(cb_hbm, idx_hbm, xd_hbm, cnt_hbm, idx_v, rows_v, ones_v, zbuf,
             hist_sh, sem):
    c = lax.axis_index("c")
    s = lax.axis_index("s")
    wid = c * 16 + s
    pltpu.sync_copy(idx_hbm.at[pl.ds(wid * NJ, NJ)], idx_v)
    handles = [
        pltpu.make_async_copy(cb_hbm.at[idx_v.at[j]], rows_v.at[j], sem)
        for j in range(NJ)
    ]
    for h in handles:
        h.start()
    for h in handles:
        h.wait()
    for j in range(NJ):
        pltpu.sync_copy(rows_v.at[j],
                        xd_hbm.at[pl.ds(wid * TOK_PER_TILE + j * 128, 128)])
    for i in range(512 // 16):
        zbuf[pl.ds(i * 16, 16)] = jnp.zeros((16,), jnp.float32)
    pltpu.sync_copy(zbuf, hist_sh.at[pl.ds(s * 512, 512)])
    for j in range(NJ):
        for i in range(128 // 16):
            ones_v[j, pl.ds(i * 16, 16)] = jnp.full((16,), 1.0, jnp.float32)
    plsc.subcore_barrier()
    for j in range(NJ):
        pltpu.sync_copy(ones_v.at[j], hist_sh.at[idx_v.at[j]], add=True)
    plsc.subcore_barrier()
    pltpu.sync_copy(hist_sh.at[pl.ds(s * 512, 512)],
                    cnt_hbm.at[c, pl.ds(s * 512, 512)])


@functools.cache
def _sc_gather_hist():
    return pl.kernel(
        _sc_body,
        out_type=[
            jax.ShapeDtypeStruct((NT, CD), jnp.float32),
            jax.ShapeDtypeStruct((2, NB), jnp.float32),
        ],
        mesh=plsc.VectorSubcoreMesh(core_axis_name="c", subcore_axis_name="s"),
        scratch_types=[
            pltpu.VMEM((NJ, 128), jnp.int32),
            pltpu.VMEM((NJ, 128, CD), jnp.float32),
            pltpu.VMEM((NJ, 128), jnp.float32),
            pltpu.VMEM((512,), jnp.float32),
            pltpu.VMEM_SHARED((NB,), jnp.float32),
            pltpu.SemaphoreType.DMA,
        ],
    )


def _finalize_body(xd_ref, x_ref, cnt_ref, out_ref, commit_ref, perp_ref,
                   acc):
    n = pl.program_id(0)
    o = jnp.transpose(xd_ref[0], (1, 0))
    out_ref[0] = o
    psum = jnp.sum((x_ref[0] - o) ** 2).reshape(1, 1)

    @pl.when(n == 0)
    def _():
        acc[...] = psum
        code_count = cnt_ref[0] + cnt_ref[1]
        total = jnp.sum(code_count)
        prob = code_count / total
        perp_ref[...] = jnp.exp(
            -jnp.sum(prob * jnp.log(prob + 1e-7))).reshape(1, 1)

    @pl.when(n > 0)
    def _():
        acc[...] = acc[...] + psum

    @pl.when(n == N - 1)
    def _():
        commit_ref[...] = acc[...] / (NT * CD)


def _stage_finalize(xd, x, cnt):
    return pl.pallas_call(
        _finalize_body,
        grid=(N,),
        in_specs=[
            pl.BlockSpec((1, T, CD), lambda n: (n, 0, 0)),
            pl.BlockSpec((1, CD, T), lambda n: (n, 0, 0)),
            pl.BlockSpec((2, NB), lambda n: (0, 0)),
        ],
        out_specs=[
            pl.BlockSpec((1, CD, T), lambda n: (n, 0, 0)),
            pl.BlockSpec((1, 1), lambda n: (0, 0)),
            pl.BlockSpec((1, 1), lambda n: (0, 0)),
        ],
        out_shape=[
            jax.ShapeDtypeStruct((N, CD, T), jnp.float32),
            jax.ShapeDtypeStruct((1, 1), jnp.float32),
            jax.ShapeDtypeStruct((1, 1), jnp.float32),
        ],
        scratch_shapes=[pltpu.VMEM((1, 1), jnp.float32)],
    )(xd, x, cnt)


def kernel(x, codebook):
    code_idx = _stage_argmin(x, codebook)
    idx2 = code_idx.reshape(NT // 128, 128)
    xd, cnt = _sc_gather_hist()(codebook, idx2)
    out, commit, perp = _stage_finalize(xd.reshape(N, T, CD), x, cnt)
    return out, commit.reshape(()), perp.reshape(())
